# baseline (speedup 1.0000x reference)
import jax
import jax.numpy as jnp
from jax import lax
from jax.experimental import pallas as pl
from jax.experimental.pallas import tpu as pltpu
from jax.experimental.pallas import tpu_sc as plsc

_NH = 12
_NN = 20736
_NU = 3312
_FLAT = _NH * _NN
_NC = 2
_CHUNK = _NN // 2
_NWORK = 2 * _NH


def _sc_gather_body(table_hbm, idx_hbm, out_hbm, table_v, idx_v, out_v):
    wid = lax.axis_index("s") * _NC + lax.axis_index("c")

    @pl.when(wid < _NWORK)
    def _():
        h = wid // 2
        c0 = (wid % 2) * _CHUNK
        pltpu.sync_copy(table_hbm, table_v)
        pltpu.sync_copy(idx_hbm.at[pl.ds(c0, _CHUNK)], idx_v)

        def body(j, carry):
            ivec = idx_v[pl.ds(j * 16, 16)] * _NH + h
            out_v[pl.ds(j * 16, 16)] = plsc.load_gather(table_v, [ivec])
            return carry

        lax.fori_loop(0, _CHUNK // 16, body, 0)
        pltpu.sync_copy(out_v, out_hbm.at[pl.ds(h * _NN + c0, _CHUNK)])


_sc_gather = pl.kernel(
    _sc_gather_body,
    out_type=jax.ShapeDtypeStruct((_FLAT,), jnp.float32),
    mesh=plsc.VectorSubcoreMesh(core_axis_name="c", subcore_axis_name="s"),
    compiler_params=pltpu.CompilerParams(needs_layout_passes=False),
    scratch_types=[
        pltpu.VMEM((_NU * _NH,), jnp.float32),
        pltpu.VMEM((_CHUNK,), jnp.int32),
        pltpu.VMEM((_CHUNK,), jnp.float32),
    ],
)


def _add_body(attn_ref, bias_ref, out_ref):
    out_ref[...] = attn_ref[...] + bias_ref[...]


def _tc_add(attn2, bias2, blk):
    b = attn2.shape[0]
    return pl.pallas_call(
        _add_body,
        grid=(b // blk,),
        in_specs=[
            pl.BlockSpec((blk, _FLAT), lambda i: (i, 0)),
            pl.BlockSpec((1, _FLAT), lambda i: (0, 0)),
        ],
        out_specs=pl.BlockSpec((blk, _FLAT), lambda i: (i, 0)),
        out_shape=jax.ShapeDtypeStruct((b, _FLAT), jnp.float32),
        compiler_params=pltpu.CompilerParams(
            dimension_semantics=("arbitrary",)),
    )(attn2, bias2)


def kernel(attn, earth_specific_bias, position_index, window_type_index):
    w = jnp.asarray(window_type_index, jnp.int32)
    table = lax.dynamic_index_in_dim(
        earth_specific_bias, w, axis=1, keepdims=False)
    bias_flat = _sc_gather(table.reshape(-1), position_index.astype(jnp.int32))
    b = attn.shape[0]
    out = _tc_add(attn.reshape(b, _FLAT), bias_flat.reshape(1, _FLAT), blk=8)
    return out.reshape(attn.shape)
